# baseline (speedup 1.0000x reference)
import functools

import jax
import jax.numpy as jnp
from jax import lax
from jax.experimental import pallas as pl
from jax.experimental.pallas import tpu as pltpu
from jax.experimental.pallas import tpu_sc as plsc

N = 10000
E = 320000
D_IN = 128
D_HID = 64
D_OUT = 128

NT = 16
EPT = E // NT
RPT = 640
NPAD = RPT * NT
DEGW = 16
RB = 1280
GRID_R = NPAD // RB

DCH = 125
DIB = 20
DNB = EPT // (DIB * DCH)

GCH = 50
GIB = 25
GNB = EPT // (GIB * GCH)
NR = 6
LA = NR - 1


def _make_deg_kernel():
    mesh = plsc.VectorSubcoreMesh(core_axis_name="c", subcore_axis_name="s")

    @functools.partial(
        pl.kernel,
        mesh=mesh,
        compiler_params=pltpu.CompilerParams(use_tc_tiling_on_sc=False),
        out_type=jax.ShapeDtypeStruct((2, NPAD, DEGW), jnp.float32),
        scratch_types=[
            pltpu.VMEM((2, DIB, DCH), jnp.int32),
            pltpu.VMEM((DCH, DEGW), jnp.float32),
            pltpu.VMEM_SHARED((NPAD, DEGW), jnp.float32),
            pltpu.SemaphoreType.DMA,
            pltpu.SemaphoreType.DMA,
        ],
    )
    def deg_kernel(dst_hbm, ones_hbm, zeros_hbm, out_hbm, dst_v, ones_v, acc, isem, ssem):
        g = lax.axis_index("c")
        s = lax.axis_index("s")
        base = s * RPT
        pltpu.async_copy(dst_hbm.at[g, s, 0], dst_v.at[0], isem)
        pltpu.sync_copy(ones_hbm, ones_v)
        pltpu.sync_copy(zeros_hbm, acc.at[pl.ds(base, RPT)])
        pltpu.make_async_copy(dst_hbm.at[g, s, 0], dst_v.at[0], isem).wait()
        plsc.subcore_barrier()

        def block(b, carry):
            bb = b % 2
            nb = (b + 1) % 2

            @pl.when(b + 1 < DNB)
            def _():
                pltpu.async_copy(dst_hbm.at[g, s, b + 1], dst_v.at[nb], isem)

            def step(c, carry2):
                pltpu.async_copy(ones_v, acc.at[dst_v.at[bb, c]], ssem, add=True)
                return carry2

            lax.fori_loop(0, DIB, step, 0)

            def drain(c, carry2):
                pltpu.make_async_copy(ones_v, acc.at[dst_v.at[bb, 0]], ssem).wait()
                return carry2

            lax.fori_loop(0, DIB, drain, 0)

            @pl.when(b + 1 < DNB)
            def _():
                pltpu.make_async_copy(dst_hbm.at[g, s, b + 1], dst_v.at[nb], isem).wait()

            return carry

        lax.fori_loop(0, DNB, block, 0)
        plsc.subcore_barrier()
        pltpu.sync_copy(acc.at[pl.ds(base, RPT)], out_hbm.at[g, pl.ds(base, RPT)])

    return deg_kernel


def _make_gs_kernel(d):
    mesh = plsc.VectorSubcoreMesh(core_axis_name="c", subcore_axis_name="s")

    @functools.partial(
        pl.kernel,
        mesh=mesh,
        compiler_params=pltpu.CompilerParams(use_tc_tiling_on_sc=False),
        out_type=jax.ShapeDtypeStruct((2, NPAD, d), jnp.float32),
        scratch_types=[
            pltpu.VMEM((2, GIB, GCH), jnp.int32),
            pltpu.VMEM((2, GIB, GCH), jnp.int32),
            pltpu.VMEM((NR, GCH, d), jnp.float32),
            pltpu.VMEM_SHARED((NPAD, d), jnp.float32),
            pltpu.SemaphoreType.DMA,
            pltpu.SemaphoreType.DMA,
            pltpu.SemaphoreType.DMA,
        ],
    )
    def gs_kernel(src_hbm, dst_hbm, y_hbm, out_hbm, src_v, dst_v, rows_v, acc, isem, gsem, ssem):
        g = lax.axis_index("c")
        s = lax.axis_index("s")
        base = s * RPT
        pltpu.async_copy(src_hbm.at[g, s, 0], src_v.at[0], isem)
        pltpu.async_copy(dst_hbm.at[g, s, 0], dst_v.at[0], isem)
        pltpu.sync_copy(y_hbm.at[pl.ds(g * NPAD + base, RPT)], acc.at[pl.ds(base, RPT)])
        pltpu.make_async_copy(src_hbm.at[g, s, 0], src_v.at[0], isem).wait()
        pltpu.make_async_copy(dst_hbm.at[g, s, 0], dst_v.at[0], isem).wait()
        plsc.subcore_barrier()
        for j in range(LA):
            pltpu.async_copy(y_hbm.at[src_v.at[0, j]], rows_v.at[j], gsem)

        def block(b, carry):
            bb = b % 2
            nb = (b + 1) % 2

            @pl.when(b + 1 < GNB)
            def _():
                pltpu.async_copy(src_hbm.at[g, s, b + 1], src_v.at[nb], isem)
                pltpu.async_copy(dst_hbm.at[g, s, b + 1], dst_v.at[nb], isem)

            def step(c, carry2):
                k = b * GIB + c
                cur = k % NR
                pre = (k + LA) % NR

                @pl.when(k >= 1)
                def _():
                    pltpu.make_async_copy(
                        rows_v.at[pre], acc.at[dst_v.at[bb, c]], ssem
                    ).wait()

                cross = jnp.logical_and(c + LA >= GIB, b + 1 < GNB)

                @pl.when(jnp.logical_and(cross, c + LA == GIB))
                def _():
                    pltpu.make_async_copy(src_hbm.at[g, s, b + 1], src_v.at[nb], isem).wait()
                    pltpu.make_async_copy(dst_hbm.at[g, s, b + 1], dst_v.at[nb], isem).wait()

                @pl.when(c + LA < GIB)
                def _():
                    pltpu.async_copy(y_hbm.at[src_v.at[bb, c + LA]], rows_v.at[pre], gsem)

                @pl.when(cross)
                def _():
                    pltpu.async_copy(y_hbm.at[src_v.at[nb, c + LA - GIB]], rows_v.at[pre], gsem)

                pltpu.make_async_copy(y_hbm.at[src_v.at[bb, c]], rows_v.at[cur], gsem).wait()
                pltpu.async_copy(rows_v.at[cur], acc.at[dst_v.at[bb, c]], ssem, add=True)
                return carry2

            lax.fori_loop(0, GIB, step, 0)
            return carry

        lax.fori_loop(0, GNB, block, 0)
        pltpu.make_async_copy(
            rows_v.at[(GNB * GIB - 1) % NR], acc.at[dst_v.at[(GNB - 1) % 2, GIB - 1]], ssem
        ).wait()
        plsc.subcore_barrier()
        pltpu.sync_copy(acc.at[pl.ds(base, RPT)], out_hbm.at[g, pl.ds(base, RPT)])

    return gs_kernel


_deg_call = _make_deg_kernel()
_gs64 = _make_gs_kernel(D_HID)
_gs128 = _make_gs_kernel(D_OUT)


def _tc_pre(x_ref, deg_ref, w_ref, y_ref):
    dinv = lax.rsqrt(deg_ref[0][:, 0:1] + 1.0)
    y_ref[0] = dinv * jnp.dot(x_ref[0], w_ref[0], preferred_element_type=jnp.float32)


def _tc_mid(s_ref, deg_ref, b_ref, w_ref, y2_ref):
    dinv = lax.rsqrt(deg_ref[0][:, 0:1] + 1.0)
    o = dinv * s_ref[0] + b_ref[0, 0]
    h = jnp.where(o > 0, o, jnp.exp(o) - 1.0)
    y2_ref[0] = dinv * jnp.dot(h, w_ref[0], preferred_element_type=jnp.float32)


def _tc_post(s_ref, deg_ref, b_ref, h_ref):
    dinv = lax.rsqrt(deg_ref[0][:, 0:1] + 1.0)
    o = dinv * s_ref[0] + b_ref[0, 0]
    h_ref[0] = jnp.where(o > 0, o, jnp.exp(o) - 1.0)


def kernel(x0, edge_index0, x1, edge_index1, W1_0, b1_0, W2_0, b2_0, W1_1, b1_1, W2_1, b2_1):
    f32 = jnp.float32
    xs = jnp.stack([
        jnp.pad(x0, ((0, NPAD - N), (0, 0))),
        jnp.pad(x1, ((0, NPAD - N), (0, 0))),
    ])
    src_flat = jnp.stack([
        edge_index0[0],
        edge_index1[0] + NPAD,
    ])
    dst_flat = jnp.stack([edge_index0[1], edge_index1[1]])
    src_g = src_flat.reshape(2, NT, GNB, GIB, GCH)
    dst_g = dst_flat.reshape(2, NT, GNB, GIB, GCH)
    dst_d = dst_flat.reshape(2, NT, DNB, DIB, DCH)
    ones = jnp.ones((DCH, DEGW), f32)
    zeros = jnp.zeros((RPT, DEGW), f32)

    deg16 = _deg_call(dst_d, ones, zeros)

    w1 = jnp.stack([W1_0, W1_1])
    w2 = jnp.stack([W2_0, W2_1])
    b1 = jnp.stack([b1_0, b1_1]).reshape(2, 1, D_HID)
    b2 = jnp.stack([b2_0, b2_1]).reshape(2, 1, D_OUT)

    y1 = pl.pallas_call(
        _tc_pre,
        grid=(2, GRID_R),
        in_specs=[
            pl.BlockSpec((1, RB, D_IN), lambda g, i: (g, i, 0)),
            pl.BlockSpec((1, RB, DEGW), lambda g, i: (g, i, 0)),
            pl.BlockSpec((1, D_IN, D_HID), lambda g, i: (g, 0, 0)),
        ],
        out_specs=pl.BlockSpec((1, RB, D_HID), lambda g, i: (g, i, 0)),
        out_shape=jax.ShapeDtypeStruct((2, NPAD, D_HID), f32),
    )(xs, deg16, w1)

    s1 = _gs64(src_g, dst_g, y1.reshape(2 * NPAD, D_HID))

    y2 = pl.pallas_call(
        _tc_mid,
        grid=(2, GRID_R),
        in_specs=[
            pl.BlockSpec((1, RB, D_HID), lambda g, i: (g, i, 0)),
            pl.BlockSpec((1, RB, DEGW), lambda g, i: (g, i, 0)),
            pl.BlockSpec((1, 1, D_HID), lambda g, i: (g, 0, 0)),
            pl.BlockSpec((1, D_HID, D_OUT), lambda g, i: (g, 0, 0)),
        ],
        out_specs=pl.BlockSpec((1, RB, D_OUT), lambda g, i: (g, i, 0)),
        out_shape=jax.ShapeDtypeStruct((2, NPAD, D_OUT), f32),
    )(s1, deg16, b1, w2)

    s2 = _gs128(src_g, dst_g, y2.reshape(2 * NPAD, D_OUT))

    h2 = pl.pallas_call(
        _tc_post,
        grid=(2, GRID_R),
        in_specs=[
            pl.BlockSpec((1, RB, D_OUT), lambda g, i: (g, i, 0)),
            pl.BlockSpec((1, RB, DEGW), lambda g, i: (g, i, 0)),
            pl.BlockSpec((1, 1, D_OUT), lambda g, i: (g, 0, 0)),
        ],
        out_specs=pl.BlockSpec((1, RB, D_OUT), lambda g, i: (g, i, 0)),
        out_shape=jax.ShapeDtypeStruct((2, NPAD, D_OUT), f32),
    )(s2, deg16, b2)

    return h2[:, :N, :].reshape(2 * N, D_OUT)


# speedup vs baseline: 1.0843x; 1.0843x over previous
import functools

import jax
import jax.numpy as jnp
from jax import lax
from jax.experimental import pallas as pl
from jax.experimental.pallas import tpu as pltpu
from jax.experimental.pallas import tpu_sc as plsc

N = 10000
E = 320000
D_IN = 128
D_HID = 64
D_OUT = 128

NT = 16
EPT = E // NT
RPT = 640
NPAD = RPT * NT
DEGW = 16
RB = 1280
GRID_R = NPAD // RB

DCH = 125
DIB = 20
DNB = EPT // (DIB * DCH)

GCH = 80
GIB = 25
GNB = EPT // (GIB * GCH)


def _make_deg_kernel():
    mesh = plsc.VectorSubcoreMesh(core_axis_name="c", subcore_axis_name="s")

    @functools.partial(
        pl.kernel,
        mesh=mesh,
        compiler_params=pltpu.CompilerParams(use_tc_tiling_on_sc=False),
        out_type=jax.ShapeDtypeStruct((2, NPAD, DEGW), jnp.float32),
        scratch_types=[
            pltpu.VMEM((2, DIB, DCH), jnp.int32),
            pltpu.VMEM((DCH, DEGW), jnp.float32),
            pltpu.VMEM_SHARED((NPAD, DEGW), jnp.float32),
            pltpu.SemaphoreType.DMA,
            pltpu.SemaphoreType.DMA,
        ],
    )
    def deg_kernel(dst_hbm, ones_hbm, zeros_hbm, out_hbm, dst_v, ones_v, acc, isem, ssem):
        g = lax.axis_index("c")
        s = lax.axis_index("s")
        base = s * RPT
        pltpu.async_copy(dst_hbm.at[g, s, 0], dst_v.at[0], isem)
        pltpu.sync_copy(ones_hbm, ones_v)
        pltpu.sync_copy(zeros_hbm, acc.at[pl.ds(base, RPT)])
        pltpu.make_async_copy(dst_hbm.at[g, s, 0], dst_v.at[0], isem).wait()
        plsc.subcore_barrier()

        def block(b, carry):
            bb = b % 2
            nb = (b + 1) % 2

            @pl.when(b + 1 < DNB)
            def _():
                pltpu.async_copy(dst_hbm.at[g, s, b + 1], dst_v.at[nb], isem)

            def step(c, carry2):
                pltpu.async_copy(ones_v, acc.at[dst_v.at[bb, c]], ssem, add=True)
                return carry2

            lax.fori_loop(0, DIB, step, 0)

            def drain(c, carry2):
                pltpu.make_async_copy(ones_v, acc.at[dst_v.at[bb, 0]], ssem).wait()
                return carry2

            lax.fori_loop(0, DIB, drain, 0)

            @pl.when(b + 1 < DNB)
            def _():
                pltpu.make_async_copy(dst_hbm.at[g, s, b + 1], dst_v.at[nb], isem).wait()

            return carry

        lax.fori_loop(0, DNB, block, 0)
        plsc.subcore_barrier()
        pltpu.sync_copy(acc.at[pl.ds(base, RPT)], out_hbm.at[g, pl.ds(base, RPT)])

    return deg_kernel


def _make_gs_kernel(d, NR):
    LA = NR - 1
    mesh = plsc.VectorSubcoreMesh(core_axis_name="c", subcore_axis_name="s")

    @functools.partial(
        pl.kernel,
        mesh=mesh,
        compiler_params=pltpu.CompilerParams(use_tc_tiling_on_sc=False),
        out_type=jax.ShapeDtypeStruct((2, NPAD, d), jnp.float32),
        scratch_types=[
            pltpu.VMEM((2, GIB, GCH), jnp.int32),
            pltpu.VMEM((2, GIB, GCH), jnp.int32),
            pltpu.VMEM((NR, GCH, d), jnp.float32),
            pltpu.VMEM_SHARED((NPAD, d), jnp.float32),
            pltpu.SemaphoreType.DMA,
            pltpu.SemaphoreType.DMA,
            pltpu.SemaphoreType.DMA,
        ],
    )
    def gs_kernel(src_hbm, dst_hbm, y_hbm, out_hbm, src_v, dst_v, rows_v, acc, isem, gsem, ssem):
        g = lax.axis_index("c")
        s = lax.axis_index("s")
        base = s * RPT
        pltpu.async_copy(src_hbm.at[g, s, 0], src_v.at[0], isem)
        pltpu.async_copy(dst_hbm.at[g, s, 0], dst_v.at[0], isem)
        pltpu.sync_copy(y_hbm.at[pl.ds(g * NPAD + base, RPT)], acc.at[pl.ds(base, RPT)])
        pltpu.make_async_copy(src_hbm.at[g, s, 0], src_v.at[0], isem).wait()
        pltpu.make_async_copy(dst_hbm.at[g, s, 0], dst_v.at[0], isem).wait()
        plsc.subcore_barrier()
        for j in range(LA):
            pltpu.async_copy(y_hbm.at[src_v.at[0, j]], rows_v.at[j], gsem)

        def block(b, carry):
            bb = b % 2
            nb = (b + 1) % 2

            @pl.when(b + 1 < GNB)
            def _():
                pltpu.async_copy(src_hbm.at[g, s, b + 1], src_v.at[nb], isem)
                pltpu.async_copy(dst_hbm.at[g, s, b + 1], dst_v.at[nb], isem)

            def step(c, carry2):
                k = b * GIB + c
                cur = k % NR
                pre = (k + LA) % NR

                @pl.when(k >= 1)
                def _():
                    pltpu.make_async_copy(
                        rows_v.at[pre], acc.at[dst_v.at[bb, c]], ssem
                    ).wait()

                cross = jnp.logical_and(c + LA >= GIB, b + 1 < GNB)

                @pl.when(jnp.logical_and(cross, c + LA == GIB))
                def _():
                    pltpu.make_async_copy(src_hbm.at[g, s, b + 1], src_v.at[nb], isem).wait()
                    pltpu.make_async_copy(dst_hbm.at[g, s, b + 1], dst_v.at[nb], isem).wait()

                @pl.when(c + LA < GIB)
                def _():
                    pltpu.async_copy(y_hbm.at[src_v.at[bb, c + LA]], rows_v.at[pre], gsem)

                @pl.when(cross)
                def _():
                    pltpu.async_copy(y_hbm.at[src_v.at[nb, c + LA - GIB]], rows_v.at[pre], gsem)

                pltpu.make_async_copy(y_hbm.at[src_v.at[bb, c]], rows_v.at[cur], gsem).wait()
                pltpu.async_copy(rows_v.at[cur], acc.at[dst_v.at[bb, c]], ssem, add=True)
                return carry2

            lax.fori_loop(0, GIB, step, 0)
            return carry

        lax.fori_loop(0, GNB, block, 0)
        pltpu.make_async_copy(
            rows_v.at[(GNB * GIB - 1) % NR], acc.at[dst_v.at[(GNB - 1) % 2, GIB - 1]], ssem
        ).wait()
        plsc.subcore_barrier()
        pltpu.sync_copy(acc.at[pl.ds(base, RPT)], out_hbm.at[g, pl.ds(base, RPT)])

    return gs_kernel


_deg_call = _make_deg_kernel()
_gs64 = _make_gs_kernel(D_HID, 8)
_gs128 = _make_gs_kernel(D_OUT, 4)


def _tc_pre(x_ref, deg_ref, w_ref, y_ref):
    dinv = lax.rsqrt(deg_ref[0][:, 0:1] + 1.0)
    y_ref[0] = dinv * jnp.dot(x_ref[0], w_ref[0], preferred_element_type=jnp.float32)


def _tc_mid(s_ref, deg_ref, b_ref, w_ref, y2_ref):
    dinv = lax.rsqrt(deg_ref[0][:, 0:1] + 1.0)
    o = dinv * s_ref[0] + b_ref[0, 0]
    h = jnp.where(o > 0, o, jnp.exp(o) - 1.0)
    y2_ref[0] = dinv * jnp.dot(h, w_ref[0], preferred_element_type=jnp.float32)


def _tc_post(s_ref, deg_ref, b_ref, h_ref):
    dinv = lax.rsqrt(deg_ref[0][:, 0:1] + 1.0)
    o = dinv * s_ref[0] + b_ref[0, 0]
    h_ref[0] = jnp.where(o > 0, o, jnp.exp(o) - 1.0)


def kernel(x0, edge_index0, x1, edge_index1, W1_0, b1_0, W2_0, b2_0, W1_1, b1_1, W2_1, b2_1):
    f32 = jnp.float32
    xs = jnp.stack([
        jnp.pad(x0, ((0, NPAD - N), (0, 0))),
        jnp.pad(x1, ((0, NPAD - N), (0, 0))),
    ])
    src_flat = jnp.stack([
        edge_index0[0],
        edge_index1[0] + NPAD,
    ])
    dst_flat = jnp.stack([edge_index0[1], edge_index1[1]])
    src_g = src_flat.reshape(2, NT, GNB, GIB, GCH)
    dst_g = dst_flat.reshape(2, NT, GNB, GIB, GCH)
    dst_d = dst_flat.reshape(2, NT, DNB, DIB, DCH)
    ones = jnp.ones((DCH, DEGW), f32)
    zeros = jnp.zeros((RPT, DEGW), f32)

    deg16 = _deg_call(dst_d, ones, zeros)

    w1 = jnp.stack([W1_0, W1_1])
    w2 = jnp.stack([W2_0, W2_1])
    b1 = jnp.stack([b1_0, b1_1]).reshape(2, 1, D_HID)
    b2 = jnp.stack([b2_0, b2_1]).reshape(2, 1, D_OUT)

    y1 = pl.pallas_call(
        _tc_pre,
        grid=(2, GRID_R),
        in_specs=[
            pl.BlockSpec((1, RB, D_IN), lambda g, i: (g, i, 0)),
            pl.BlockSpec((1, RB, DEGW), lambda g, i: (g, i, 0)),
            pl.BlockSpec((1, D_IN, D_HID), lambda g, i: (g, 0, 0)),
        ],
        out_specs=pl.BlockSpec((1, RB, D_HID), lambda g, i: (g, i, 0)),
        out_shape=jax.ShapeDtypeStruct((2, NPAD, D_HID), f32),
    )(xs, deg16, w1)

    s1 = _gs64(src_g, dst_g, y1.reshape(2 * NPAD, D_HID))

    y2 = pl.pallas_call(
        _tc_mid,
        grid=(2, GRID_R),
        in_specs=[
            pl.BlockSpec((1, RB, D_HID), lambda g, i: (g, i, 0)),
            pl.BlockSpec((1, RB, DEGW), lambda g, i: (g, i, 0)),
            pl.BlockSpec((1, 1, D_HID), lambda g, i: (g, 0, 0)),
            pl.BlockSpec((1, D_HID, D_OUT), lambda g, i: (g, 0, 0)),
        ],
        out_specs=pl.BlockSpec((1, RB, D_OUT), lambda g, i: (g, i, 0)),
        out_shape=jax.ShapeDtypeStruct((2, NPAD, D_OUT), f32),
    )(s1, deg16, b1, w2)

    s2 = _gs128(src_g, dst_g, y2.reshape(2 * NPAD, D_OUT))

    h2 = pl.pallas_call(
        _tc_post,
        grid=(2, GRID_R),
        in_specs=[
            pl.BlockSpec((1, RB, D_OUT), lambda g, i: (g, i, 0)),
            pl.BlockSpec((1, RB, DEGW), lambda g, i: (g, i, 0)),
            pl.BlockSpec((1, 1, D_OUT), lambda g, i: (g, 0, 0)),
        ],
        out_specs=pl.BlockSpec((1, RB, D_OUT), lambda g, i: (g, i, 0)),
        out_shape=jax.ShapeDtypeStruct((2, NPAD, D_OUT), f32),
    )(s2, deg16, b2)

    return h2[:, :N, :].reshape(2 * N, D_OUT)
